# baseline (speedup 1.0000x reference)
import functools

import jax
import jax.numpy as jnp
from jax.experimental import pallas as pl
from jax.experimental.pallas import tpu as pltpu


def _body(x_ref, h_ref, h0_ref, sel_ref, anc_ref, out_ref, *, A: int, M: int):
    xb = x_ref[...]
    parts = jax.lax.dot_general(
        xb.astype(jnp.bfloat16), h_ref[...].astype(jnp.bfloat16),
        (((2,), (2,)), ((1,), (1,))),
        preferred_element_type=jnp.float32)
    modes = jnp.sum(parts, axis=0)
    h0parts = jax.lax.dot_general(
        xb, h0_ref[...], (((2,), (2,)), ((1,), (1,))),
        preferred_element_type=jnp.float32)
    h0col = jnp.sum(h0parts, axis=0)

    mmax = jnp.max(modes, axis=1, keepdims=True)
    lse = mmax + jnp.log(jnp.sum(jnp.exp(modes - mmax), axis=1, keepdims=True))
    mn = modes - lse

    sel_all = sel_ref[...]
    row_ids = jax.lax.broadcasted_iota(jnp.int32, sel_all.shape, 0)

    acc = jnp.zeros((xb.shape[0], 1), dtype=jnp.float32)
    for a in range(A):
        idx = anc_ref[a]
        sel_row = jnp.sum(jnp.where(row_ids == idx, sel_all, 0.0), axis=0,
                          keepdims=True)
        t = mn + jnp.log(sel_row)
        tmax = jnp.max(t, axis=1, keepdims=True)
        acc = acc + tmax + jnp.log(
            jnp.sum(jnp.exp(t - tmax), axis=1, keepdims=True))

    out_ref[...] = -(acc + h0col)


@jax.jit
def kernel(x, h0, H, selected_modes, ancestors):
    N, L, Q = x.shape
    M = H.shape[0]
    T = selected_modes.shape[0]
    A = ancestors.shape[0]

    x3 = x.transpose(0, 2, 1)
    h3 = H.transpose(0, 2, 1)
    h03 = h0.T[None]
    sel = selected_modes.astype(jnp.float32)
    anc = ancestors.astype(jnp.int32)

    BN = 512
    grid = (N // BN,)
    out = pl.pallas_call(
        functools.partial(_body, A=A, M=M),
        grid=grid,
        in_specs=[
            pl.BlockSpec((BN, Q, L), lambda i: (i, 0, 0)),
            pl.BlockSpec((M, Q, L), lambda i: (0, 0, 0)),
            pl.BlockSpec((1, Q, L), lambda i: (0, 0, 0)),
            pl.BlockSpec((T, M), lambda i: (0, 0)),
            pl.BlockSpec(memory_space=pltpu.SMEM),
        ],
        out_specs=pl.BlockSpec((BN, 1), lambda i: (i, 0)),
        out_shape=jax.ShapeDtypeStruct((N, 1), jnp.float32),
        compiler_params=pltpu.CompilerParams(
            dimension_semantics=("parallel",)),
    )(x3, h3, h03, sel, anc)
    return out.reshape(N)


# speedup vs baseline: 1.4418x; 1.4418x over previous
import functools

import jax
import jax.numpy as jnp
from jax.experimental import pallas as pl
from jax.experimental.pallas import tpu as pltpu


def _body(x_ref, h_ref, h0_ref, sel_ref, anc_ref, out_ref, *, A: int, M: int):
    xb = x_ref[...]
    parts = jax.lax.dot_general(
        xb.astype(jnp.bfloat16), h_ref[...].astype(jnp.bfloat16),
        (((2,), (2,)), ((1,), (1,))),
        preferred_element_type=jnp.float32)
    modes = jnp.sum(parts, axis=0)
    h0col = jnp.sum(xb * h0_ref[...], axis=(1, 2))[:, None]

    mmax = jnp.max(modes, axis=1, keepdims=True)
    lse = mmax + jnp.log(jnp.sum(jnp.exp(modes - mmax), axis=1, keepdims=True))
    mn = modes - lse

    sel_all = sel_ref[...]
    row_ids = jax.lax.broadcasted_iota(jnp.int32, sel_all.shape, 0)

    acc = jnp.zeros((xb.shape[0], 1), dtype=jnp.float32)
    for a in range(A):
        idx = anc_ref[a]
        sel_row = jnp.sum(jnp.where(row_ids == idx, sel_all, 0.0), axis=0,
                          keepdims=True)
        t = mn + jnp.log(sel_row)
        tmax = jnp.max(t, axis=1, keepdims=True)
        acc = acc + tmax + jnp.log(
            jnp.sum(jnp.exp(t - tmax), axis=1, keepdims=True))

    out_ref[...] = -(acc + h0col)


@jax.jit
def kernel(x, h0, H, selected_modes, ancestors):
    N, L, Q = x.shape
    M = H.shape[0]
    T = selected_modes.shape[0]
    A = ancestors.shape[0]

    x3 = x.transpose(0, 2, 1)
    h3 = H.transpose(0, 2, 1)
    h03 = h0.T[None]
    sel = selected_modes.astype(jnp.float32)
    anc = ancestors.astype(jnp.int32)

    BN = 512
    grid = (N // BN,)
    out = pl.pallas_call(
        functools.partial(_body, A=A, M=M),
        grid=grid,
        in_specs=[
            pl.BlockSpec((BN, Q, L), lambda i: (i, 0, 0)),
            pl.BlockSpec((M, Q, L), lambda i: (0, 0, 0)),
            pl.BlockSpec((1, Q, L), lambda i: (0, 0, 0)),
            pl.BlockSpec((T, M), lambda i: (0, 0)),
            pl.BlockSpec(memory_space=pltpu.SMEM),
        ],
        out_specs=pl.BlockSpec((BN, 1), lambda i: (i, 0)),
        out_shape=jax.ShapeDtypeStruct((N, 1), jnp.float32),
        compiler_params=pltpu.CompilerParams(
            dimension_semantics=("parallel",)),
    )(x3, h3, h03, sel, anc)
    return out.reshape(N)
